# baseline (speedup 1.0000x reference)
import functools
import math

import numpy as np
import jax
import jax.numpy as jnp
from jax import lax
from jax.experimental import pallas as pl
from jax.experimental.pallas import tpu as pltpu
from jax.experimental.pallas import tpu_sc as plsc

VOCAB = 100000
D = 256
L_SEQ = 200
BATCH = 1024
SEQ_OUT = L_SEQ + 2
NLANE = 16
NV = D // NLANE
NW = 32
CHUNK = 128
CHUNKS_PER_L = BATCH // CHUNK
N_MAIN = L_SEQ * CHUNKS_PER_L
MAIN_PER_W = N_MAIN // NW
TAIL_ROWS = 2 * BATCH
TAIL_PER_W = TAIL_ROWS // NW


def _pos_encoding_np(max_len, d_model):
    position = np.arange(0, max_len).astype(np.float32)[:, None]
    div_term = np.exp(
        np.arange(0, d_model, 2).astype(np.float32) * -(math.log(10000.0) / d_model)
    )
    pe = np.zeros((max_len, d_model), dtype=np.float32)
    pe[:, 0::2] = np.sin(position * div_term)
    pe[:, 1::2] = np.cos(position * div_term)
    return pe


_PE_NP = _pos_encoding_np(1000, D)[:L_SEQ]
_DEST_MAIN_NP = (
    np.arange(BATCH, dtype=np.int32)[None, :] * SEQ_OUT
    + np.arange(L_SEQ, dtype=np.int32)[:, None]
)
_DEST_TAIL_NP = (
    np.arange(BATCH, dtype=np.int32)[None, :] * SEQ_OUT
    + np.array([L_SEQ, L_SEQ + 1], dtype=np.int32)[:, None]
).reshape(-1)


def _mlp_body(num_ref, sd_ref, nw1_ref, nb1_ref, nw2_ref, nb2_ref,
              sw1_ref, sb1_ref, sw2_ref, sb2_ref, out_ref):
    hi = jax.lax.Precision.HIGHEST
    h = jnp.maximum(
        jnp.dot(num_ref[...], nw1_ref[...], precision=hi) + nb1_ref[...], 0.0)
    ne = jnp.dot(h, nw2_ref[...], precision=hi) + nb2_ref[...]
    hs = jnp.maximum(
        jnp.dot(sd_ref[...], sw1_ref[...], precision=hi) + sb1_ref[...], 0.0)
    se = jnp.dot(hs, sw2_ref[...], precision=hi) + sb2_ref[...]
    out_ref[0, :, :] = ne
    out_ref[1, :, :] = se


def _mlp_encodings(num_in, sd, nw1, nb1, nw2, nb2, sw1, sb1, sw2, sb2):
    return pl.pallas_call(
        _mlp_body,
        out_shape=jax.ShapeDtypeStruct((2, BATCH, D), jnp.float32),
    )(num_in, sd, nw1, nb1, nw2, nb2, sw1, sb1, sw2, sb2)


@functools.partial(
    pl.kernel,
    mesh=plsc.VectorSubcoreMesh(core_axis_name="c", subcore_axis_name="s"),
    out_type=jax.ShapeDtypeStruct((BATCH * SEQ_OUT, D), jnp.float32),
    scratch_types=[
        pltpu.VMEM((CHUNK,), jnp.int32),
        pltpu.VMEM((CHUNK,), jnp.int32),
        pltpu.VMEM((D,), jnp.float32),
        pltpu.VMEM((CHUNK, D), jnp.float32),
        pltpu.VMEM((TAIL_PER_W,), jnp.int32),
        pltpu.VMEM((TAIL_PER_W, D), jnp.float32),
        pltpu.SemaphoreType.DMA,
    ],
)
def _sc_encode(ids_t_hbm, dest_hbm, pe_hbm, table_hbm, enc2_hbm, dest2_hbm,
               out_hbm, sidx_v, didx_v, pe_v, rows_v, didx2_v, rows2_v, sem):
    wid = lax.axis_index("s") * 2 + lax.axis_index("c")

    @pl.loop(0, MAIN_PER_W)
    def _(t):
        g = wid * MAIN_PER_W + t
        l = g // CHUNKS_PER_L
        b0 = (g % CHUNKS_PER_L) * CHUNK
        pltpu.sync_copy(ids_t_hbm.at[l, pl.ds(b0, CHUNK)], sidx_v)
        pltpu.sync_copy(dest_hbm.at[l, pl.ds(b0, CHUNK)], didx_v)
        pltpu.sync_copy(pe_hbm.at[l], pe_v)
        pltpu.async_copy(table_hbm.at[sidx_v], rows_v, sem).wait()
        pe_regs = [pe_v[pl.ds(NLANE * j, NLANE)] for j in range(NV)]

        @pl.loop(0, CHUNK)
        def _(r):
            for j in range(NV):
                plsc.addupdate(rows_v.at[r, pl.ds(NLANE * j, NLANE)], pe_regs[j])

        pltpu.async_copy(rows_v, out_hbm.at[didx_v], sem).wait()

    base = wid * TAIL_PER_W
    pltpu.sync_copy(dest2_hbm.at[pl.ds(base, TAIL_PER_W)], didx2_v)
    pltpu.sync_copy(enc2_hbm.at[pl.ds(base, TAIL_PER_W)], rows2_v)
    pltpu.async_copy(rows2_v, out_hbm.at[didx2_v], sem).wait()


def kernel(text_ids, numerical_data, structured_data, table,
           nw1, nb1, nw2, nb2, sw1, sb1, sw2, sb2):
    ids_t = text_ids.astype(jnp.int32).T
    enc2 = _mlp_encodings(
        numerical_data.reshape(BATCH, 1), structured_data,
        nw1, nb1.reshape(1, -1), nw2, nb2.reshape(1, -1),
        sw1, sb1.reshape(1, -1), sw2, sb2.reshape(1, -1))
    out_flat = _sc_encode(
        ids_t,
        jnp.asarray(_DEST_MAIN_NP),
        jnp.asarray(_PE_NP),
        table,
        enc2.reshape(TAIL_ROWS, D),
        jnp.asarray(_DEST_TAIL_NP),
    )
    return out_flat.reshape(BATCH, SEQ_OUT, D)


# speedup vs baseline: 1.3048x; 1.3048x over previous
import functools
import math

import numpy as np
import jax
import jax.numpy as jnp
from jax import lax
from jax.experimental import pallas as pl
from jax.experimental.pallas import tpu as pltpu
from jax.experimental.pallas import tpu_sc as plsc

VOCAB = 100000
D = 256
L_SEQ = 200
BATCH = 1024
SEQ_OUT = L_SEQ + 2
NLANE = 16
NV = D // NLANE
NW = 32
CHUNK = 128
CHUNKS_PER_L = BATCH // CHUNK
N_MAIN = L_SEQ * CHUNKS_PER_L
MAIN_PER_W = N_MAIN // NW
NTRIPLES = 48 // 3
PE_ROWS = 16
PE_PAD = 208
TAIL_ROWS = 2 * BATCH
TAIL_PER_W = TAIL_ROWS // NW


def _pos_encoding_np(max_len, d_model):
    position = np.arange(0, max_len).astype(np.float32)[:, None]
    div_term = np.exp(
        np.arange(0, d_model, 2).astype(np.float32) * -(math.log(10000.0) / d_model)
    )
    pe = np.zeros((max_len, d_model), dtype=np.float32)
    pe[:, 0::2] = np.sin(position * div_term)
    pe[:, 1::2] = np.cos(position * div_term)
    return pe


_PE_NP = np.zeros((PE_PAD, D), dtype=np.float32)
_PE_NP[:L_SEQ] = _pos_encoding_np(1000, D)[:L_SEQ]
_DEST_MAIN_NP = (
    np.arange(BATCH, dtype=np.int32)[None, :] * SEQ_OUT
    + np.arange(L_SEQ, dtype=np.int32)[:, None]
).reshape(NW, MAIN_PER_W, CHUNK)
_DEST_TAIL_NP = (
    np.arange(BATCH, dtype=np.int32)[None, :] * SEQ_OUT
    + np.array([L_SEQ, L_SEQ + 1], dtype=np.int32)[:, None]
).reshape(-1)


def _mlp_body(num_ref, sd_ref, nw1_ref, nb1_ref, nw2_ref, nb2_ref,
              sw1_ref, sb1_ref, sw2_ref, sb2_ref, out_ref):
    hi = jax.lax.Precision.HIGHEST
    h = jnp.maximum(
        jnp.dot(num_ref[...], nw1_ref[...], precision=hi) + nb1_ref[...], 0.0)
    ne = jnp.dot(h, nw2_ref[...], precision=hi) + nb2_ref[...]
    hs = jnp.maximum(
        jnp.dot(sd_ref[...], sw1_ref[...], precision=hi) + sb1_ref[...], 0.0)
    se = jnp.dot(hs, sw2_ref[...], precision=hi) + sb2_ref[...]
    out_ref[0, :, :] = ne
    out_ref[1, :, :] = se


def _mlp_encodings(num_in, sd, nw1, nb1, nw2, nb2, sw1, sb1, sw2, sb2):
    return pl.pallas_call(
        _mlp_body,
        out_shape=jax.ShapeDtypeStruct((2, BATCH, D), jnp.float32),
    )(num_in, sd, nw1, nb1, nw2, nb2, sw1, sb1, sw2, sb2)


@functools.partial(
    pl.kernel,
    mesh=plsc.VectorSubcoreMesh(core_axis_name="c", subcore_axis_name="s"),
    out_type=jax.ShapeDtypeStruct((BATCH * SEQ_OUT, D), jnp.float32),
    scratch_types=[
        pltpu.VMEM((MAIN_PER_W, CHUNK), jnp.int32),
        pltpu.VMEM((MAIN_PER_W, CHUNK), jnp.int32),
        pltpu.VMEM((PE_ROWS, D), jnp.float32),
        pltpu.VMEM((CHUNK, D), jnp.float32),
        pltpu.VMEM((CHUNK, D), jnp.float32),
        pltpu.VMEM((CHUNK, D), jnp.float32),
        pltpu.VMEM((TAIL_PER_W,), jnp.int32),
        pltpu.SemaphoreType.DMA,
        pltpu.SemaphoreType.DMA,
        pltpu.SemaphoreType.DMA,
        pltpu.SemaphoreType.DMA,
        pltpu.SemaphoreType.DMA,
        pltpu.SemaphoreType.DMA,
    ],
)
def _sc_encode(ids_hbm, dest_hbm, pe_hbm, table_hbm, enc2_hbm, dest2_hbm,
               out_hbm, sidx_v, didx_v, pe_v, rows0, rows1, rows2, didx2_v,
               g0, g1, g2, s0, s1, s2):
    wid = lax.axis_index("s") * 2 + lax.axis_index("c")
    rows = (rows0, rows1, rows2)
    gsem = (g0, g1, g2)
    ssem = (s0, s1, s2)
    lo = (wid * MAIN_PER_W) // CHUNKS_PER_L
    lo8 = pl.multiple_of((lo // 8) * 8, 8)

    pltpu.sync_copy(ids_hbm.at[wid], sidx_v)
    pltpu.sync_copy(dest_hbm.at[wid], didx_v)
    pltpu.sync_copy(pe_hbm.at[pl.ds(lo8, PE_ROWS)], pe_v)

    def gather_issue(t, p):
        pltpu.async_copy(table_hbm.at[sidx_v.at[t]], rows[p], gsem[p])

    def gather_wait(p):
        pltpu.make_async_copy(table_hbm.at[sidx_v.at[0]], rows[p], gsem[p]).wait()

    def scatter_issue(t, p):
        pltpu.async_copy(rows[p], out_hbm.at[didx_v.at[t]], ssem[p])

    def scatter_wait(p):
        pltpu.make_async_copy(rows[p], out_hbm.at[didx_v.at[0]], ssem[p]).wait()

    def pe_add(t, p):
        rowi = (wid * MAIN_PER_W + t) // CHUNKS_PER_L - lo8
        pe_regs = [pe_v[rowi, pl.ds(NLANE * j, NLANE)] for j in range(NV)]
        buf = rows[p]

        @pl.loop(0, CHUNK)
        def _(r):
            for j in range(NV):
                plsc.addupdate(buf.at[r, pl.ds(NLANE * j, NLANE)], pe_regs[j])

    gather_issue(0, 0)
    gather_issue(1, 1)

    @pl.loop(0, NTRIPLES)
    def _(k):
        t0 = k * 3
        for ph in range(3):
            t = t0 + ph
            pn = (ph + 2) % 3
            gather_wait(ph)
            pe_add(t, ph)
            scatter_issue(t, ph)
            if ph == 0:
                @pl.when(k >= 1)
                def _():
                    scatter_wait(pn)
            else:
                scatter_wait(pn)
            gather_issue(t + 2, pn)

    for t, p in ((48, 0), (49, 1)):
        gather_wait(p)
        pe_add(t, p)
        scatter_issue(t, p)
    scatter_wait(2)
    scatter_wait(0)
    scatter_wait(1)

    base = wid * TAIL_PER_W
    tail_rows = rows0.at[pl.ds(0, TAIL_PER_W)]
    pltpu.sync_copy(dest2_hbm.at[pl.ds(base, TAIL_PER_W)], didx2_v)
    pltpu.sync_copy(enc2_hbm.at[pl.ds(base, TAIL_PER_W)], tail_rows)
    pltpu.async_copy(tail_rows, out_hbm.at[didx2_v], g0).wait()


def kernel(text_ids, numerical_data, structured_data, table,
           nw1, nb1, nw2, nb2, sw1, sb1, sw2, sb2):
    ids_flat = text_ids.astype(jnp.int32).T.reshape(NW, MAIN_PER_W, CHUNK)
    enc2 = _mlp_encodings(
        numerical_data.reshape(BATCH, 1), structured_data,
        nw1, nb1.reshape(1, -1), nw2, nb2.reshape(1, -1),
        sw1, sb1.reshape(1, -1), sw2, sb2.reshape(1, -1))
    out_flat = _sc_encode(
        ids_flat,
        jnp.asarray(_DEST_MAIN_NP),
        jnp.asarray(_PE_NP),
        table,
        enc2.reshape(TAIL_ROWS, D),
        jnp.asarray(_DEST_TAIL_NP),
    )
    return out_flat.reshape(BATCH, SEQ_OUT, D)


# speedup vs baseline: 3.5322x; 2.7071x over previous
import functools
import math

import numpy as np
import jax
import jax.numpy as jnp
from jax import lax
from jax.experimental import pallas as pl
from jax.experimental.pallas import tpu as pltpu
from jax.experimental.pallas import tpu_sc as plsc

VOCAB = 100000
D = 256
L_SEQ = 200
BATCH = 1024
SEQ_OUT = L_SEQ + 2
NLANE = 16
NV = D // NLANE
NW = 32
CHUNK = 128
CHUNKS_PER_L = BATCH // CHUNK
N_MAIN = L_SEQ * CHUNKS_PER_L
MAIN_PER_W = N_MAIN // NW
NTRIPLES = 48 // 3
PE_ROWS = 16
PE_PAD = 208
TAIL_ROWS = 2 * BATCH
TAIL_PER_W = TAIL_ROWS // NW


def _pos_encoding_np(max_len, d_model):
    position = np.arange(0, max_len).astype(np.float32)[:, None]
    div_term = np.exp(
        np.arange(0, d_model, 2).astype(np.float32) * -(math.log(10000.0) / d_model)
    )
    pe = np.zeros((max_len, d_model), dtype=np.float32)
    pe[:, 0::2] = np.sin(position * div_term)
    pe[:, 1::2] = np.cos(position * div_term)
    return pe


_PE_NP = np.zeros((PE_PAD, D), dtype=np.float32)
_PE_NP[:L_SEQ] = _pos_encoding_np(1000, D)[:L_SEQ]


def _mlp_body(num_ref, sd_ref, nw1_ref, nb1_ref, nw2_ref, nb2_ref,
              sw1_ref, sb1_ref, sw2_ref, sb2_ref, out_ref):
    hi = jax.lax.Precision.HIGHEST
    h = jnp.maximum(
        jnp.dot(num_ref[...], nw1_ref[...], precision=hi) + nb1_ref[...], 0.0)
    ne = jnp.dot(h, nw2_ref[...], precision=hi) + nb2_ref[...]
    hs = jnp.maximum(
        jnp.dot(sd_ref[...], sw1_ref[...], precision=hi) + sb1_ref[...], 0.0)
    se = jnp.dot(hs, sw2_ref[...], precision=hi) + sb2_ref[...]
    out_ref[0, :, :] = ne
    out_ref[1, :, :] = se


def _mlp_encodings(num_in, sd, nw1, nb1, nw2, nb2, sw1, sb1, sw2, sb2):
    return pl.pallas_call(
        _mlp_body,
        out_shape=jax.ShapeDtypeStruct((2, BATCH, D), jnp.float32),
    )(num_in, sd, nw1, nb1, nw2, nb2, sw1, sb1, sw2, sb2)


@functools.partial(
    pl.kernel,
    mesh=plsc.VectorSubcoreMesh(core_axis_name="c", subcore_axis_name="s"),
    out_type=jax.ShapeDtypeStruct((SEQ_OUT * BATCH, D), jnp.float32),
    scratch_types=[
        pltpu.VMEM((MAIN_PER_W, CHUNK), jnp.int32),
        pltpu.VMEM((PE_ROWS, D), jnp.float32),
        pltpu.VMEM((CHUNK, D), jnp.float32),
        pltpu.VMEM((CHUNK, D), jnp.float32),
        pltpu.VMEM((CHUNK, D), jnp.float32),
        pltpu.SemaphoreType.DMA,
        pltpu.SemaphoreType.DMA,
        pltpu.SemaphoreType.DMA,
        pltpu.SemaphoreType.DMA,
        pltpu.SemaphoreType.DMA,
        pltpu.SemaphoreType.DMA,
    ],
)
def _sc_encode(ids_hbm, pe_hbm, table_hbm, enc2_hbm, out_hbm,
               sidx_v, pe_v, rows0, rows1, rows2, g0, g1, g2, s0, s1, s2):
    wid = lax.axis_index("s") * 2 + lax.axis_index("c")
    rows = (rows0, rows1, rows2)
    gsem = (g0, g1, g2)
    ssem = (s0, s1, s2)
    lo = (wid * MAIN_PER_W) // CHUNKS_PER_L
    lo8 = pl.multiple_of((lo // 8) * 8, 8)

    pltpu.sync_copy(ids_hbm.at[wid], sidx_v)
    pltpu.sync_copy(pe_hbm.at[pl.ds(lo8, PE_ROWS)], pe_v)

    def gather_issue(t, p):
        pltpu.async_copy(table_hbm.at[sidx_v.at[t]], rows[p], gsem[p])

    def gather_wait(p):
        pltpu.make_async_copy(table_hbm.at[sidx_v.at[0]], rows[p], gsem[p]).wait()

    def store_issue(t, p):
        row0 = pl.multiple_of((wid * MAIN_PER_W + t) * CHUNK, CHUNK)
        pltpu.async_copy(rows[p], out_hbm.at[pl.ds(row0, CHUNK)], ssem[p])

    def store_wait(p):
        pltpu.make_async_copy(rows[p], out_hbm.at[pl.ds(0, CHUNK)], ssem[p]).wait()

    def pe_add(t, p):
        rowi = (wid * MAIN_PER_W + t) // CHUNKS_PER_L - lo8
        pe_regs = [pe_v[rowi, pl.ds(NLANE * j, NLANE)] for j in range(NV)]
        buf = rows[p]

        @pl.loop(0, CHUNK)
        def _(r):
            for j in range(NV):
                plsc.addupdate(buf.at[r, pl.ds(NLANE * j, NLANE)], pe_regs[j])

    gather_issue(0, 0)
    gather_issue(1, 1)

    @pl.loop(0, NTRIPLES)
    def _(k):
        t0 = k * 3
        for ph in range(3):
            t = t0 + ph
            pn = (ph + 2) % 3
            gather_wait(ph)
            pe_add(t, ph)
            store_issue(t, ph)
            if ph == 0:
                @pl.when(k >= 1)
                def _():
                    store_wait(pn)
            else:
                store_wait(pn)
            gather_issue(t + 2, pn)

    for t, p in ((48, 0), (49, 1)):
        gather_wait(p)
        pe_add(t, p)
        store_issue(t, p)
    store_wait(2)
    store_wait(0)
    store_wait(1)

    base = wid * TAIL_PER_W
    tail_buf = rows0.at[pl.ds(0, TAIL_PER_W)]
    pltpu.sync_copy(enc2_hbm.at[pl.ds(base, TAIL_PER_W)], tail_buf)
    pltpu.sync_copy(tail_buf, out_hbm.at[pl.ds(L_SEQ * BATCH + base, TAIL_PER_W)])


def kernel(text_ids, numerical_data, structured_data, table,
           nw1, nb1, nw2, nb2, sw1, sb1, sw2, sb2):
    ids_lmajor = text_ids.astype(jnp.int32).T.reshape(NW, MAIN_PER_W, CHUNK)
    enc2 = _mlp_encodings(
        numerical_data.reshape(BATCH, 1), structured_data,
        nw1, nb1.reshape(1, -1), nw2, nb2.reshape(1, -1),
        sw1, sb1.reshape(1, -1), sw2, sb2.reshape(1, -1))
    out_flat = _sc_encode(
        ids_lmajor,
        jnp.asarray(_PE_NP),
        table,
        enc2.reshape(TAIL_ROWS, D),
    )
    return jnp.swapaxes(out_flat.reshape(SEQ_OUT, BATCH, D), 0, 1)


# speedup vs baseline: 3.6217x; 1.0253x over previous
import functools
import math

import numpy as np
import jax
import jax.numpy as jnp
from jax import lax
from jax.experimental import pallas as pl
from jax.experimental.pallas import tpu as pltpu
from jax.experimental.pallas import tpu_sc as plsc

VOCAB = 100000
D = 256
L_SEQ = 200
BATCH = 1024
SEQ_OUT = L_SEQ + 2
NLANE = 16
NV = D // NLANE
NW = 32
CHUNK = 128
CHUNKS_PER_L = BATCH // CHUNK
N_MAIN = L_SEQ * CHUNKS_PER_L
MAIN_PER_W = N_MAIN // NW
NTRIPLES = 48 // 3
PE_ROWS = 16
PE_PAD = 208
TAIL_ROWS = 2 * BATCH
TAIL_PER_W = TAIL_ROWS // NW


def _pos_encoding_np(max_len, d_model):
    position = np.arange(0, max_len).astype(np.float32)[:, None]
    div_term = np.exp(
        np.arange(0, d_model, 2).astype(np.float32) * -(math.log(10000.0) / d_model)
    )
    pe = np.zeros((max_len, d_model), dtype=np.float32)
    pe[:, 0::2] = np.sin(position * div_term)
    pe[:, 1::2] = np.cos(position * div_term)
    return pe


_PE_NP = np.zeros((PE_PAD, D), dtype=np.float32)
_PE_NP[:L_SEQ] = _pos_encoding_np(1000, D)[:L_SEQ]


def _mlp_body(buf_ref, num_ref, sd_ref, nw1_ref, nb1_ref, nw2_ref, nb2_ref,
              sw1_ref, sb1_ref, sw2_ref, sb2_ref, out_ref):
    del buf_ref
    hi = jax.lax.Precision.HIGHEST
    h = jnp.maximum(
        jnp.dot(num_ref[...], nw1_ref[...], precision=hi) + nb1_ref[...], 0.0)
    ne = jnp.dot(h, nw2_ref[...], precision=hi) + nb2_ref[...]
    hs = jnp.maximum(
        jnp.dot(sd_ref[...], sw1_ref[...], precision=hi) + sb1_ref[...], 0.0)
    se = jnp.dot(hs, sw2_ref[...], precision=hi) + sb2_ref[...]
    out_ref[:BATCH, :] = ne
    out_ref[BATCH:, :] = se


def _write_mlp_tail(out_flat, num_in, sd, nw1, nb1, nw2, nb2, sw1, sb1, sw2, sb2):
    tail_spec = pl.BlockSpec((TAIL_ROWS, D),
                             lambda i: (L_SEQ * BATCH // TAIL_ROWS, 0))
    full = lambda s: pl.BlockSpec(s, lambda i, _s=s: tuple(0 for _ in _s))
    return pl.pallas_call(
        _mlp_body,
        grid=(1,),
        in_specs=[
            tail_spec,
            full((BATCH, 1)), full((BATCH, 10)),
            full((1, 64)), full((1, 64)), full((64, D)), full((1, D)),
            full((10, 128)), full((1, 128)), full((128, D)), full((1, D)),
        ],
        out_specs=tail_spec,
        out_shape=jax.ShapeDtypeStruct((SEQ_OUT * BATCH, D), jnp.float32),
        input_output_aliases={0: 0},
    )(out_flat, num_in, sd, nw1, nb1, nw2, nb2, sw1, sb1, sw2, sb2)


@functools.partial(
    pl.kernel,
    mesh=plsc.VectorSubcoreMesh(core_axis_name="c", subcore_axis_name="s"),
    out_type=jax.ShapeDtypeStruct((SEQ_OUT * BATCH, D), jnp.float32),
    scratch_types=[
        pltpu.VMEM((MAIN_PER_W, CHUNK), jnp.int32),
        pltpu.VMEM((PE_ROWS, D), jnp.float32),
        pltpu.VMEM((CHUNK, D), jnp.float32),
        pltpu.VMEM((CHUNK, D), jnp.float32),
        pltpu.VMEM((CHUNK, D), jnp.float32),
        pltpu.SemaphoreType.DMA,
        pltpu.SemaphoreType.DMA,
        pltpu.SemaphoreType.DMA,
        pltpu.SemaphoreType.DMA,
        pltpu.SemaphoreType.DMA,
        pltpu.SemaphoreType.DMA,
    ],
)
def _sc_encode(ids_hbm, pe_hbm, table_hbm, out_hbm,
               sidx_v, pe_v, rows0, rows1, rows2, g0, g1, g2, s0, s1, s2):
    wid = lax.axis_index("s") * 2 + lax.axis_index("c")
    rows = (rows0, rows1, rows2)
    gsem = (g0, g1, g2)
    ssem = (s0, s1, s2)
    lo = (wid * MAIN_PER_W) // CHUNKS_PER_L
    lo8 = pl.multiple_of((lo // 8) * 8, 8)

    pltpu.sync_copy(ids_hbm.at[wid], sidx_v)
    pltpu.sync_copy(pe_hbm.at[pl.ds(lo8, PE_ROWS)], pe_v)

    def gather_issue(t, p):
        pltpu.async_copy(table_hbm.at[sidx_v.at[t]], rows[p], gsem[p])

    def gather_wait(p):
        pltpu.make_async_copy(table_hbm.at[sidx_v.at[0]], rows[p], gsem[p]).wait()

    def store_issue(t, p):
        row0 = pl.multiple_of((wid * MAIN_PER_W + t) * CHUNK, CHUNK)
        pltpu.async_copy(rows[p], out_hbm.at[pl.ds(row0, CHUNK)], ssem[p])

    def store_wait(p):
        pltpu.make_async_copy(rows[p], out_hbm.at[pl.ds(0, CHUNK)], ssem[p]).wait()

    def pe_add(t, p):
        rowi = (wid * MAIN_PER_W + t) // CHUNKS_PER_L - lo8
        pe_regs = [pe_v[rowi, pl.ds(NLANE * j, NLANE)] for j in range(NV)]
        buf = rows[p]

        @pl.loop(0, CHUNK, step=2)
        def _(r):
            for rr in range(2):
                for j in range(NV):
                    plsc.addupdate(buf.at[r + rr, pl.ds(NLANE * j, NLANE)],
                                   pe_regs[j])

    gather_issue(0, 0)
    gather_issue(1, 1)

    @pl.loop(0, NTRIPLES)
    def _(k):
        t0 = k * 3
        for ph in range(3):
            t = t0 + ph
            pn = (ph + 2) % 3
            gather_wait(ph)
            pe_add(t, ph)
            store_issue(t, ph)
            if ph == 0:
                @pl.when(k >= 1)
                def _():
                    store_wait(pn)
            else:
                store_wait(pn)
            gather_issue(t + 2, pn)

    for t, p in ((48, 0), (49, 1)):
        gather_wait(p)
        pe_add(t, p)
        store_issue(t, p)
    store_wait(2)
    store_wait(0)
    store_wait(1)


def kernel(text_ids, numerical_data, structured_data, table,
           nw1, nb1, nw2, nb2, sw1, sb1, sw2, sb2):
    ids_lmajor = text_ids.astype(jnp.int32).T.reshape(NW, MAIN_PER_W, CHUNK)
    out_flat = _sc_encode(
        ids_lmajor,
        jnp.asarray(_PE_NP),
        table,
    )
    out_flat = _write_mlp_tail(
        out_flat,
        numerical_data.reshape(BATCH, 1), structured_data,
        nw1, nb1.reshape(1, -1), nw2, nb2.reshape(1, -1),
        sw1, sb1.reshape(1, -1), sw2, sb2.reshape(1, -1))
    return jnp.swapaxes(out_flat.reshape(SEQ_OUT, BATCH, D), 0, 1)


# speedup vs baseline: 3.6264x; 1.0013x over previous
import functools
import math

import numpy as np
import jax
import jax.numpy as jnp
from jax import lax
from jax.experimental import pallas as pl
from jax.experimental.pallas import tpu as pltpu
from jax.experimental.pallas import tpu_sc as plsc

VOCAB = 100000
D = 256
L_SEQ = 200
BATCH = 1024
SEQ_OUT = L_SEQ + 2
NLANE = 16
NV = D // NLANE
NW = 32
CHUNK = 64
CHUNKS_PER_L = BATCH // CHUNK
N_MAIN = L_SEQ * CHUNKS_PER_L
MAIN_PER_W = N_MAIN // NW
NBUF = 5
NROUNDS = MAIN_PER_W // NBUF
PE_ROWS = 16
PE_PAD = 208
TAIL_ROWS = 2 * BATCH
TAIL_PER_W = TAIL_ROWS // NW


def _pos_encoding_np(max_len, d_model):
    position = np.arange(0, max_len).astype(np.float32)[:, None]
    div_term = np.exp(
        np.arange(0, d_model, 2).astype(np.float32) * -(math.log(10000.0) / d_model)
    )
    pe = np.zeros((max_len, d_model), dtype=np.float32)
    pe[:, 0::2] = np.sin(position * div_term)
    pe[:, 1::2] = np.cos(position * div_term)
    return pe


_PE_NP = np.zeros((PE_PAD, D), dtype=np.float32)
_PE_NP[:L_SEQ] = _pos_encoding_np(1000, D)[:L_SEQ]


def _mlp_body(buf_ref, num_ref, sd_ref, nw1_ref, nb1_ref, nw2_ref, nb2_ref,
              sw1_ref, sb1_ref, sw2_ref, sb2_ref, out_ref):
    del buf_ref
    hi = jax.lax.Precision.HIGHEST
    h = jnp.maximum(
        jnp.dot(num_ref[...], nw1_ref[...], precision=hi) + nb1_ref[...], 0.0)
    ne = jnp.dot(h, nw2_ref[...], precision=hi) + nb2_ref[...]
    hs = jnp.maximum(
        jnp.dot(sd_ref[...], sw1_ref[...], precision=hi) + sb1_ref[...], 0.0)
    se = jnp.dot(hs, sw2_ref[...], precision=hi) + sb2_ref[...]
    out_ref[:BATCH, :] = ne
    out_ref[BATCH:, :] = se


def _write_mlp_tail(out_flat, num_in, sd, nw1, nb1, nw2, nb2, sw1, sb1, sw2, sb2):
    tail_spec = pl.BlockSpec((TAIL_ROWS, D),
                             lambda i: (L_SEQ * BATCH // TAIL_ROWS, 0))
    full = lambda s: pl.BlockSpec(s, lambda i, _s=s: tuple(0 for _ in _s))
    return pl.pallas_call(
        _mlp_body,
        grid=(1,),
        in_specs=[
            tail_spec,
            full((BATCH, 1)), full((BATCH, 10)),
            full((1, 64)), full((1, 64)), full((64, D)), full((1, D)),
            full((10, 128)), full((1, 128)), full((128, D)), full((1, D)),
        ],
        out_specs=tail_spec,
        out_shape=jax.ShapeDtypeStruct((SEQ_OUT * BATCH, D), jnp.float32),
        input_output_aliases={0: 0},
    )(out_flat, num_in, sd, nw1, nb1, nw2, nb2, sw1, sb1, sw2, sb2)


@functools.partial(
    pl.kernel,
    mesh=plsc.VectorSubcoreMesh(core_axis_name="c", subcore_axis_name="s"),
    out_type=jax.ShapeDtypeStruct((SEQ_OUT * BATCH, D), jnp.float32),
    scratch_types=[
        pltpu.VMEM((MAIN_PER_W, CHUNK), jnp.int32),
        pltpu.VMEM((PE_ROWS, D), jnp.float32),
        *([pltpu.VMEM((CHUNK, D), jnp.float32)] * NBUF),
        *([pltpu.SemaphoreType.DMA] * NBUF),
        *([pltpu.SemaphoreType.DMA] * NBUF),
    ],
)
def _sc_encode(ids_hbm, pe_hbm, table_hbm, out_hbm, sidx_v, pe_v, *bufs):
    wid = lax.axis_index("s") * 2 + lax.axis_index("c")
    rows = bufs[:NBUF]
    gsem = bufs[NBUF:2 * NBUF]
    ssem = bufs[2 * NBUF:]
    lo = (wid * MAIN_PER_W) // CHUNKS_PER_L
    lo8 = pl.multiple_of((lo // 8) * 8, 8)

    pltpu.sync_copy(ids_hbm.at[wid], sidx_v)
    pltpu.sync_copy(pe_hbm.at[pl.ds(lo8, PE_ROWS)], pe_v)

    def gather_issue(t, p):
        pltpu.async_copy(table_hbm.at[sidx_v.at[t]], rows[p], gsem[p])

    def gather_wait(p):
        pltpu.make_async_copy(table_hbm.at[sidx_v.at[0]], rows[p], gsem[p]).wait()

    def store_issue(t, p):
        row0 = pl.multiple_of((wid * MAIN_PER_W + t) * CHUNK, CHUNK)
        pltpu.async_copy(rows[p], out_hbm.at[pl.ds(row0, CHUNK)], ssem[p])

    def store_wait(p):
        pltpu.make_async_copy(rows[p], out_hbm.at[pl.ds(0, CHUNK)], ssem[p]).wait()

    def pe_add(t, p):
        rowi = (wid * MAIN_PER_W + t) // CHUNKS_PER_L - lo8
        pe_regs = [pe_v[rowi, pl.ds(NLANE * j, NLANE)] for j in range(NV)]
        buf = rows[p]

        @pl.loop(0, CHUNK, step=2)
        def _(r):
            for rr in range(2):
                for j in range(NV):
                    plsc.addupdate(buf.at[r + rr, pl.ds(NLANE * j, NLANE)],
                                   pe_regs[j])

    for t in range(NBUF - 2):
        gather_issue(t, t)

    @pl.loop(0, NROUNDS)
    def _(k):
        t0 = k * NBUF
        for ph in range(NBUF):
            t = t0 + ph
            pn = (ph + NBUF - 2) % NBUF
            gather_wait(ph)
            pe_add(t, ph)
            store_issue(t, ph)
            @pl.when(t + NBUF - 2 < MAIN_PER_W)
            def _():
                @pl.when(t >= 2)
                def _():
                    store_wait(pn)
                gather_issue(t + NBUF - 2, pn)

    for p in range(NBUF):
        store_wait(p)


def kernel(text_ids, numerical_data, structured_data, table,
           nw1, nb1, nw2, nb2, sw1, sb1, sw2, sb2):
    ids_lmajor = text_ids.astype(jnp.int32).T.reshape(NW, MAIN_PER_W, CHUNK)
    out_flat = _sc_encode(
        ids_lmajor,
        jnp.asarray(_PE_NP),
        table,
    )
    out_flat = _write_mlp_tail(
        out_flat,
        numerical_data.reshape(BATCH, 1), structured_data,
        nw1, nb1.reshape(1, -1), nw2, nb2.reshape(1, -1),
        sw1, sb1.reshape(1, -1), sw2, sb2.reshape(1, -1))
    return jnp.swapaxes(out_flat.reshape(SEQ_OUT, BATCH, D), 0, 1)


# speedup vs baseline: 3.7157x; 1.0246x over previous
import functools
import math

import numpy as np
import jax
import jax.numpy as jnp
from jax import lax
from jax.experimental import pallas as pl
from jax.experimental.pallas import tpu as pltpu
from jax.experimental.pallas import tpu_sc as plsc

VOCAB = 100000
D = 256
L_SEQ = 200
BATCH = 1024
SEQ_OUT = L_SEQ + 2
NLANE = 16
NV = D // NLANE
NW = 32
CHUNK = 64
CHUNKS_PER_L = BATCH // CHUNK
N_MAIN = L_SEQ * CHUNKS_PER_L
MAIN_PER_W = N_MAIN // NW
NBUF = 5
NROUNDS = MAIN_PER_W // NBUF
PE_ROWS = 16
PE_PAD = 208
TAIL_ROWS = 2 * BATCH
TAIL_PER_W = TAIL_ROWS // NW


def _pos_encoding_np(max_len, d_model):
    position = np.arange(0, max_len).astype(np.float32)[:, None]
    div_term = np.exp(
        np.arange(0, d_model, 2).astype(np.float32) * -(math.log(10000.0) / d_model)
    )
    pe = np.zeros((max_len, d_model), dtype=np.float32)
    pe[:, 0::2] = np.sin(position * div_term)
    pe[:, 1::2] = np.cos(position * div_term)
    return pe


_PE_NP = np.zeros((PE_PAD, D), dtype=np.float32)
_PE_NP[:L_SEQ] = _pos_encoding_np(1000, D)[:L_SEQ]


def _mlp_body(buf_ref, num_ref, sd_ref, nw1_ref, nb1_ref, nw2_ref, nb2_ref,
              sw1_ref, sb1_ref, sw2_ref, sb2_ref, out_ref):
    del buf_ref
    h = jnp.maximum(
        jnp.dot(num_ref[...], nw1_ref[...],
                preferred_element_type=jnp.float32) + nb1_ref[...], 0.0)
    ne = jnp.dot(h, nw2_ref[...],
                 preferred_element_type=jnp.float32) + nb2_ref[...]
    hs = jnp.maximum(
        jnp.dot(sd_ref[...], sw1_ref[...],
                preferred_element_type=jnp.float32) + sb1_ref[...], 0.0)
    se = jnp.dot(hs, sw2_ref[...],
                 preferred_element_type=jnp.float32) + sb2_ref[...]
    out_ref[:BATCH, :] = ne
    out_ref[BATCH:, :] = se


_IDS_ROWS = 16


def _write_mlp_tail(out_flat, num_in, sd, nw1, nb1, nw2, nb2, sw1, sb1, sw2, sb2):
    tail_spec = pl.BlockSpec((TAIL_ROWS, D),
                             lambda i: (L_SEQ * BATCH // TAIL_ROWS, 0))
    full = lambda s: pl.BlockSpec(s, lambda i, _s=s: tuple(0 for _ in _s))
    return pl.pallas_call(
        _mlp_body,
        grid=(1,),
        in_specs=[
            pl.BlockSpec(memory_space=pl.ANY),
            full((BATCH, 1)), full((BATCH, 10)),
            full((1, 64)), full((1, 64)), full((64, D)), full((1, D)),
            full((10, 128)), full((1, 128)), full((128, D)), full((1, D)),
        ],
        out_specs=tail_spec,
        out_shape=jax.ShapeDtypeStruct((SEQ_OUT * BATCH, D), jnp.float32),
        input_output_aliases={0: 0},
    )(out_flat, num_in, sd, nw1, nb1, nw2, nb2, sw1, sb1, sw2, sb2)


@functools.partial(
    pl.kernel,
    mesh=plsc.VectorSubcoreMesh(core_axis_name="c", subcore_axis_name="s"),
    out_type=jax.ShapeDtypeStruct((SEQ_OUT * BATCH, D), jnp.float32),
    scratch_types=[
        pltpu.VMEM((_IDS_ROWS, BATCH), jnp.int32),
        pltpu.VMEM((PE_ROWS, D), jnp.float32),
        *([pltpu.VMEM((CHUNK, D), jnp.float32)] * NBUF),
        *([pltpu.SemaphoreType.DMA] * NBUF),
        *([pltpu.SemaphoreType.DMA] * NBUF),
    ],
)
def _sc_encode(ids_hbm, pe_hbm, table_hbm, out_hbm, sidx_v, pe_v, *bufs):
    wid = lax.axis_index("s") * 2 + lax.axis_index("c")
    rows = bufs[:NBUF]
    gsem = bufs[NBUF:2 * NBUF]
    ssem = bufs[2 * NBUF:]
    lo = (wid * MAIN_PER_W) // CHUNKS_PER_L
    lo8 = pl.multiple_of(
        lax.min((lo // 8) * 8, jnp.int32(L_SEQ - _IDS_ROWS)), 8)

    pltpu.sync_copy(ids_hbm.at[pl.ds(lo8, _IDS_ROWS)], sidx_v)
    pltpu.sync_copy(pe_hbm.at[pl.ds(lo8, PE_ROWS)], pe_v)

    def idx_slice(t):
        g = wid * MAIN_PER_W + t
        return sidx_v.at[g // CHUNKS_PER_L - lo8,
                         pl.ds((g % CHUNKS_PER_L) * CHUNK, CHUNK)]

    def gather_issue(t, p):
        pltpu.async_copy(table_hbm.at[idx_slice(t)], rows[p], gsem[p])

    def gather_wait(p):
        pltpu.make_async_copy(table_hbm.at[idx_slice(0)], rows[p], gsem[p]).wait()

    def store_issue(t, p):
        row0 = pl.multiple_of((wid * MAIN_PER_W + t) * CHUNK, CHUNK)
        pltpu.async_copy(rows[p], out_hbm.at[pl.ds(row0, CHUNK)], ssem[p])

    def store_wait(p):
        pltpu.make_async_copy(rows[p], out_hbm.at[pl.ds(0, CHUNK)], ssem[p]).wait()

    def pe_add(t, p):
        rowi = (wid * MAIN_PER_W + t) // CHUNKS_PER_L - lo8
        pe_regs = [pe_v[rowi, pl.ds(NLANE * j, NLANE)] for j in range(NV)]
        buf = rows[p]

        @pl.loop(0, CHUNK, step=2)
        def _(r):
            for rr in range(2):
                for j in range(NV):
                    plsc.addupdate(buf.at[r + rr, pl.ds(NLANE * j, NLANE)],
                                   pe_regs[j])

    for t in range(NBUF - 2):
        gather_issue(t, t)

    @pl.loop(0, NROUNDS)
    def _(k):
        t0 = k * NBUF
        for ph in range(NBUF):
            t = t0 + ph
            pn = (ph + NBUF - 2) % NBUF
            gather_wait(ph)
            pe_add(t, ph)
            store_issue(t, ph)
            @pl.when(t + NBUF - 2 < MAIN_PER_W)
            def _():
                @pl.when(t >= 2)
                def _():
                    store_wait(pn)
                gather_issue(t + NBUF - 2, pn)

    for p in range(NBUF):
        store_wait(p)


def kernel(text_ids, numerical_data, structured_data, table,
           nw1, nb1, nw2, nb2, sw1, sb1, sw2, sb2):
    ids_lmajor = text_ids.astype(jnp.int32).T
    out_flat = _sc_encode(
        ids_lmajor,
        jnp.asarray(_PE_NP),
        table,
    )
    out_flat = _write_mlp_tail(
        out_flat,
        numerical_data.reshape(BATCH, 1), structured_data,
        nw1, nb1.reshape(1, -1), nw2, nb2.reshape(1, -1),
        sw1, sb1.reshape(1, -1), sw2, sb2.reshape(1, -1))
    return jnp.swapaxes(out_flat.reshape(SEQ_OUT, BATCH, D), 0, 1)


# speedup vs baseline: 3.7287x; 1.0035x over previous
import functools
import math

import numpy as np
import jax
import jax.numpy as jnp
from jax import lax
from jax.experimental import pallas as pl
from jax.experimental.pallas import tpu as pltpu
from jax.experimental.pallas import tpu_sc as plsc

VOCAB = 100000
D = 256
L_SEQ = 200
BATCH = 1024
SEQ_OUT = L_SEQ + 2
NLANE = 16
NV = D // NLANE
NW = 32
CHUNK = 64
CHUNKS_PER_L = BATCH // CHUNK
N_MAIN = L_SEQ * CHUNKS_PER_L
MAIN_PER_W = N_MAIN // NW
NBUF = 4
NROUNDS = MAIN_PER_W // NBUF
PE_ROWS = 16
PE_PAD = 208
TAIL_ROWS = 2 * BATCH
TAIL_PER_W = TAIL_ROWS // NW


def _pos_encoding_np(max_len, d_model):
    position = np.arange(0, max_len).astype(np.float32)[:, None]
    div_term = np.exp(
        np.arange(0, d_model, 2).astype(np.float32) * -(math.log(10000.0) / d_model)
    )
    pe = np.zeros((max_len, d_model), dtype=np.float32)
    pe[:, 0::2] = np.sin(position * div_term)
    pe[:, 1::2] = np.cos(position * div_term)
    return pe


_PE_NP = np.zeros((PE_PAD, D), dtype=np.float32)
_PE_NP[:L_SEQ] = _pos_encoding_np(1000, D)[:L_SEQ]


def _mlp_body(buf_ref, num_ref, sd_ref, nw1_ref, nb1_ref, nw2_ref, nb2_ref,
              sw1_ref, sb1_ref, sw2_ref, sb2_ref, out_ref):
    del buf_ref
    h = jnp.maximum(
        jnp.dot(num_ref[...], nw1_ref[...],
                preferred_element_type=jnp.float32) + nb1_ref[...], 0.0)
    ne = jnp.dot(h, nw2_ref[...],
                 preferred_element_type=jnp.float32) + nb2_ref[...]
    hs = jnp.maximum(
        jnp.dot(sd_ref[...], sw1_ref[...],
                preferred_element_type=jnp.float32) + sb1_ref[...], 0.0)
    se = jnp.dot(hs, sw2_ref[...],
                 preferred_element_type=jnp.float32) + sb2_ref[...]
    out_ref[:BATCH, :] = ne
    out_ref[BATCH:, :] = se


_IDS_ROWS = 16


def _write_mlp_tail(out_flat, num_in, sd, nw1, nb1, nw2, nb2, sw1, sb1, sw2, sb2):
    tail_spec = pl.BlockSpec((TAIL_ROWS, D),
                             lambda i: (L_SEQ * BATCH // TAIL_ROWS, 0))
    full = lambda s: pl.BlockSpec(s, lambda i, _s=s: tuple(0 for _ in _s))
    return pl.pallas_call(
        _mlp_body,
        grid=(1,),
        in_specs=[
            pl.BlockSpec(memory_space=pl.ANY),
            full((BATCH, 1)), full((BATCH, 10)),
            full((1, 64)), full((1, 64)), full((64, D)), full((1, D)),
            full((10, 128)), full((1, 128)), full((128, D)), full((1, D)),
        ],
        out_specs=tail_spec,
        out_shape=jax.ShapeDtypeStruct((SEQ_OUT * BATCH, D), jnp.float32),
        input_output_aliases={0: 0},
    )(out_flat, num_in, sd, nw1, nb1, nw2, nb2, sw1, sb1, sw2, sb2)


@functools.partial(
    pl.kernel,
    mesh=plsc.VectorSubcoreMesh(core_axis_name="c", subcore_axis_name="s"),
    out_type=jax.ShapeDtypeStruct((SEQ_OUT * BATCH, D), jnp.float32),
    scratch_types=[
        pltpu.VMEM((_IDS_ROWS, BATCH), jnp.int32),
        pltpu.VMEM((PE_ROWS, D), jnp.float32),
        *([pltpu.VMEM((CHUNK, D), jnp.float32)] * NBUF),
        *([pltpu.SemaphoreType.DMA] * NBUF),
        *([pltpu.SemaphoreType.DMA] * NBUF),
    ],
)
def _sc_encode(ids_hbm, pe_hbm, table_hbm, out_hbm, sidx_v, pe_v, *bufs):
    wid = lax.axis_index("s") * 2 + lax.axis_index("c")
    rows = bufs[:NBUF]
    gsem = bufs[NBUF:2 * NBUF]
    ssem = bufs[2 * NBUF:]
    lo = (wid * MAIN_PER_W) // CHUNKS_PER_L
    lo8 = pl.multiple_of(
        lax.min((lo // 8) * 8, jnp.int32(L_SEQ - _IDS_ROWS)), 8)

    pltpu.sync_copy(ids_hbm.at[pl.ds(lo8, _IDS_ROWS)], sidx_v)
    pltpu.sync_copy(pe_hbm.at[pl.ds(lo8, PE_ROWS)], pe_v)

    def idx_slice(t):
        g = wid * MAIN_PER_W + t
        return sidx_v.at[g // CHUNKS_PER_L - lo8,
                         pl.ds((g % CHUNKS_PER_L) * CHUNK, CHUNK)]

    def gather_issue(t, p):
        pltpu.async_copy(table_hbm.at[idx_slice(t)], rows[p], gsem[p])

    def gather_wait(p):
        pltpu.make_async_copy(table_hbm.at[idx_slice(0)], rows[p], gsem[p]).wait()

    def store_issue(t, p):
        row0 = pl.multiple_of((wid * MAIN_PER_W + t) * CHUNK, CHUNK)
        pltpu.async_copy(rows[p], out_hbm.at[pl.ds(row0, CHUNK)], ssem[p])

    def store_wait(p):
        pltpu.make_async_copy(rows[p], out_hbm.at[pl.ds(0, CHUNK)], ssem[p]).wait()

    def pe_add(t, p):
        rowi = (wid * MAIN_PER_W + t) // CHUNKS_PER_L - lo8
        pe_regs = [pe_v[rowi, pl.ds(NLANE * j, NLANE)] for j in range(NV)]
        buf = rows[p]

        @pl.loop(0, CHUNK)
        def _(r):
            for j in range(NV):
                plsc.addupdate(buf.at[r, pl.ds(NLANE * j, NLANE)], pe_regs[j])

    for t in range(NBUF - 2):
        gather_issue(t, t)

    @pl.loop(0, NROUNDS)
    def _(k):
        t0 = k * NBUF
        for ph in range(NBUF):
            t = t0 + ph
            pn = (ph + NBUF - 2) % NBUF
            gather_wait(ph)
            pe_add(t, ph)
            store_issue(t, ph)
            @pl.when(t + NBUF - 2 < MAIN_PER_W)
            def _():
                @pl.when(t >= 2)
                def _():
                    store_wait(pn)
                gather_issue(t + NBUF - 2, pn)

    for p in range(NBUF):
        store_wait(p)


def kernel(text_ids, numerical_data, structured_data, table,
           nw1, nb1, nw2, nb2, sw1, sb1, sw2, sb2):
    ids_lmajor = text_ids.astype(jnp.int32).T
    out_flat = _sc_encode(
        ids_lmajor,
        jnp.asarray(_PE_NP),
        table,
    )
    out_flat = _write_mlp_tail(
        out_flat,
        numerical_data.reshape(BATCH, 1), structured_data,
        nw1, nb1.reshape(1, -1), nw2, nb2.reshape(1, -1),
        sw1, sb1.reshape(1, -1), sw2, sb2.reshape(1, -1))
    return jnp.swapaxes(out_flat.reshape(SEQ_OUT, BATCH, D), 0, 1)
